# baseline (speedup 1.0000x reference)
import functools

import jax
import jax.numpy as jnp
from jax import lax
from jax.experimental import pallas as pl
from jax.experimental.pallas import tpu as pltpu
from jax.experimental.pallas import tpu_sc as plsc

_N_EDGES = 320000
_HID = 128
_D2 = _HID // 2
_NW = 32
_EPW = _N_EDGES // _NW
_C = 80
_NCHUNK = _EPW // _C
_GROUPS = _C // 16


def _edge_score_body(node_hbm, rel_hbm, src_hbm, dst_hbm, ridx_hbm, out_hbm,
                     src_v, dst_v, ridx_v, out_v,
                     head_a, tail_a, rel_a, head_b, tail_b, rel_b,
                     sem_a, sem_b):
    cid = lax.axis_index("c")
    sid = lax.axis_index("s")
    wid = sid * 2 + cid
    base = wid * _EPW

    pltpu.sync_copy(src_hbm.at[pl.ds(base, _EPW)], src_v)
    pltpu.sync_copy(dst_hbm.at[pl.ds(base, _EPW)], dst_v)
    pltpu.sync_copy(ridx_hbm.at[pl.ds(base, _EPW)], ridx_v)

    def start(c, head, tail, rel, sem):
        sl = pl.ds(c * _C, _C)
        pltpu.async_copy(node_hbm.at[src_v.at[sl]], head, sem)
        pltpu.async_copy(node_hbm.at[dst_v.at[sl]], tail, sem)
        pltpu.async_copy(rel_hbm.at[ridx_v.at[sl]], rel, sem)

    def wait(c, head, tail, rel, sem):
        sl = pl.ds(c * _C, _C)
        pltpu.make_async_copy(node_hbm.at[src_v.at[sl]], head, sem).wait()
        pltpu.make_async_copy(node_hbm.at[dst_v.at[sl]], tail, sem).wait()
        pltpu.make_async_copy(rel_hbm.at[ridx_v.at[sl]], rel, sem).wait()

    lane = lax.iota(jnp.int32, 16)

    def compute(c, head_v, tail_v, rel_v):
        def group_body(g, carry2):
            vec = jnp.zeros((16,), jnp.float32)
            for j in range(16):
                k = g * 16 + j
                acc = jnp.zeros((16,), jnp.float32)
                for q in range(_D2 // 16):
                    lo = q * 16
                    hi = _D2 + q * 16
                    h_i = head_v[k, pl.ds(lo, 16)]
                    h_j = head_v[k, pl.ds(hi, 16)]
                    t_i = tail_v[k, pl.ds(lo, 16)]
                    t_j = tail_v[k, pl.ds(hi, 16)]
                    r_f = rel_v[k, pl.ds(lo, 16)]
                    r_b = rel_v[k, pl.ds(hi, 16)]
                    acc = acc + h_i * r_f * t_j + t_i * r_b * h_j
                s = jnp.full((16,), jnp.sum(acc))
                vec = jnp.where(lane == j, s, vec)
            out_v[pl.ds(c * _C + g * 16, 16)] = jnp.clip(0.5 * vec, -20.0, 20.0)
            return carry2

        lax.fori_loop(0, _GROUPS, group_body, 0)

    start(0, head_a, tail_a, rel_a, sem_a)
    start(1, head_b, tail_b, rel_b, sem_b)
    wait(0, head_a, tail_a, rel_a, sem_a)
    compute(0, head_a, tail_a, rel_a)
    start(2, head_a, tail_a, rel_a, sem_a)

    def pair_body(i, carry):
        c_b = 1 + 2 * i
        c_a = 2 + 2 * i
        wait(c_b, head_b, tail_b, rel_b, sem_b)
        compute(c_b, head_b, tail_b, rel_b)

        @pl.when(c_b + 2 < _NCHUNK)
        def _():
            start(c_b + 2, head_b, tail_b, rel_b, sem_b)

        wait(c_a, head_a, tail_a, rel_a, sem_a)
        compute(c_a, head_a, tail_a, rel_a)

        @pl.when(c_a + 2 < _NCHUNK)
        def _():
            start(c_a + 2, head_a, tail_a, rel_a, sem_a)

        return carry

    lax.fori_loop(0, (_NCHUNK - 1) // 2, pair_body, 0)
    pltpu.sync_copy(out_v, out_hbm.at[pl.ds(base, _EPW)])


@jax.jit
def _sc_edge_score(node_emb, rel_emb, src, dst, rel_idx):
    mesh = plsc.VectorSubcoreMesh(core_axis_name="c", subcore_axis_name="s")
    run = pl.kernel(
        _edge_score_body,
        mesh=mesh,
        compiler_params=pltpu.CompilerParams(needs_layout_passes=False),
        out_type=jax.ShapeDtypeStruct((_N_EDGES,), jnp.float32),
        scratch_types=[
            pltpu.VMEM((_EPW,), jnp.int32),
            pltpu.VMEM((_EPW,), jnp.int32),
            pltpu.VMEM((_EPW,), jnp.int32),
            pltpu.VMEM((_EPW,), jnp.float32),
            pltpu.VMEM((_C, _HID), jnp.float32),
            pltpu.VMEM((_C, _HID), jnp.float32),
            pltpu.VMEM((_C, _HID), jnp.float32),
            pltpu.VMEM((_C, _HID), jnp.float32),
            pltpu.VMEM((_C, _HID), jnp.float32),
            pltpu.VMEM((_C, _HID), jnp.float32),
            pltpu.SemaphoreType.DMA,
            pltpu.SemaphoreType.DMA,
        ],
    )
    return run(node_emb, rel_emb, src, dst, rel_idx)


def kernel(node_emb, rel_emb, src, dst, rel_idx):
    return _sc_edge_score(node_emb, rel_emb,
                          src.astype(jnp.int32), dst.astype(jnp.int32),
                          rel_idx.astype(jnp.int32))
